# baseline (speedup 1.0000x reference)
import functools

import jax
import jax.numpy as jnp
from jax import lax
from jax.experimental import pallas as pl
from jax.experimental.pallas import tpu as pltpu
from jax.experimental.pallas import tpu_sc as plsc

DELAY = 2048
NUM_ENVS = 16384

_NC = 2
_NS = 16
_NW = _NC * _NS
_BPW = NUM_ENVS // _NW
_CHUNK = 128
_NCHUNK = _BPW // _CHUNK
_LANES = 16


def _gather_body(i_hbm, buf_hbm, out_hbm, iraw_v, idx_v, vals_v, sem):
    wid = lax.axis_index("s") * _NC + lax.axis_index("c")
    base = wid * _BPW

    pltpu.sync_copy(i_hbm.at[wid], iraw_v)

    lane = lax.iota(jnp.int32, 16) * DELAY
    for j in range(_NCHUNK):
        for t in range(_CHUNK // _LANES):
            sl = pl.ds(t * _LANES, _LANES)
            row0 = (base + j * _CHUNK + t * _LANES) * DELAY
            idx_v[j, sl] = iraw_v[j, sl] + (lane + row0)

    cps = [
        pltpu.async_copy(buf_hbm.at[idx_v.at[j]], vals_v.at[j], sem)
        for j in range(_NCHUNK)
    ]
    for cp in cps:
        cp.wait()

    pltpu.sync_copy(vals_v, out_hbm.at[wid])


@functools.partial(
    pl.kernel,
    mesh=plsc.VectorSubcoreMesh(core_axis_name="c", subcore_axis_name="s"),
    out_type=jax.ShapeDtypeStruct((_NW, _NCHUNK, _CHUNK), jnp.float32),
    scratch_types=[
        pltpu.VMEM((_NCHUNK, _CHUNK), jnp.int32),
        pltpu.VMEM((_NCHUNK, _CHUNK), jnp.int32),
        pltpu.VMEM((_NCHUNK, _CHUNK), jnp.float32),
        pltpu.SemaphoreType.DMA,
    ],
)
def _sc_gather(i_hbm, buf_hbm, out_hbm, iraw_v, idx_v, vals_v, sem):
    _gather_body(i_hbm, buf_hbm, out_hbm, iraw_v, idx_v, vals_v, sem)


def kernel(x, buffor, i):
    del x
    i3 = i.reshape(_NW, _NCHUNK, _CHUNK)
    buf_flat = buffor.reshape(-1)
    out = _sc_gather(i3, buf_flat)
    return out.reshape(NUM_ENVS)
